# baseline (speedup 1.0000x reference)
import jax
import jax.numpy as jnp
from jax import lax
from jax.experimental import pallas as pl
from jax.experimental.pallas import tpu as pltpu
from jax.experimental.pallas import tpu_sc as plsc

N_NODES = 10000
N_EDGES = 320000
D = 128
B_IDX = 5000

NS = 16
EC = 80
N_ECHUNK = 252
E_PER_TILE = N_ECHUNK * EC
ZR = 40
N_ZCHUNK = N_NODES // ZR
N_GCHUNK = 62


def _sc_body(x_hbm, e1_hbm, w1_hbm, e2_hbm, w2_hbm, idx_hbm,
             g1_hbm, g2_hbm,
             acc, zbuf, eb0, eb1, eb2, eb3, wb0, wb1, wb2, wb3,
             rows0, rows1, rows2,
             sem, semI0, semI1, semI2, semI3, semG0, semG1, semG2, semS):
    c = lax.axis_index("c")
    s = lax.axis_index("s")
    ebs = (eb0, eb1, eb2, eb3)
    wbs = (wb0, wb1, wb2, wb3)
    rowss = (rows0, rows1, rows2)
    semIs = (semI0, semI1, semI2, semI3)
    semGs = (semG0, semG1, semG2)

    zero = jnp.zeros((16,), jnp.float32)

    def zrow(j, carry):
        for k in range(8):
            zbuf[j, pl.ds(k * 16, 16)] = zero
        return carry

    lax.fori_loop(0, ZR, zrow, 0)
    for j in range((N_ZCHUNK + NS - 1) // NS):
        m = s + NS * j

        @pl.when(m < N_ZCHUNK)
        def _():
            pltpu.sync_copy(zbuf, acc.at[pl.ds(m * ZR, ZR)])

    plsc.subcore_barrier()

    def spmm(e_hbm, w_hbm):
        base = s * N_ECHUNK
        wbase = s * E_PER_TILE

        def start_idx(j, u):
            pltpu.async_copy(e_hbm.at[base + j], ebs[u], semIs[u])
            pltpu.async_copy(
                w_hbm.at[pl.ds(wbase + j * EC, EC)], wbs[u], semIs[u])

        def wait_idx(u):
            pltpu.make_async_copy(e_hbm.at[base], ebs[u], semIs[u]).wait()
            pltpu.make_async_copy(
                w_hbm.at[pl.ds(wbase, EC)], wbs[u], semIs[u]).wait()

        def start_gather(r, e):
            pltpu.async_copy(x_hbm.at[ebs[e].at[0]], rowss[r], semGs[r])

        def wait_gather(u):
            pltpu.make_async_copy(
                x_hbm.at[pl.ds(0, EC)], rowss[u], semGs[u]).wait()

        def wait_scatter():
            pltpu.make_async_copy(
                x_hbm.at[pl.ds(0, EC)], rowss[0], semS).wait()

        def sub(j, u):
            ru = u % 3
            r2 = (u + 2) % 3
            e0 = u % 4
            e2 = (u + 2) % 4
            e3 = (u + 3) % 4

            @pl.when(j >= 1)
            def _():
                wait_scatter()

            @pl.when(j + 2 < N_ECHUNK)
            def _():
                wait_idx(e2)
                start_gather(r2, e2)

            wait_gather(ru)

            def scale16(g, carry2):
                wv = wbs[e0][pl.ds(g * 16, 16)]
                for e16 in range(16):
                    e = g * 16 + e16
                    w = wv[e16]
                    for k in range(8):
                        sl = pl.ds(k * 16, 16)
                        rowss[ru][e, sl] = rowss[ru][e, sl] * w
                return carry2

            lax.fori_loop(0, EC // 16, scale16, 0)

            pltpu.async_copy(rowss[ru], acc.at[ebs[e0].at[1]], semS, add=True)

            @pl.when(j + 3 < N_ECHUNK)
            def _():
                start_idx(j + 3, e3)

        start_idx(0, 0)
        start_idx(1, 1)
        start_idx(2, 2)
        wait_idx(0)
        start_gather(0, 0)
        wait_idx(1)
        start_gather(1, 1)

        def twelve(p, carry):
            for u in range(12):
                sub(12 * p + u, u)
            return carry

        lax.fori_loop(0, N_ECHUNK // 12, twelve, 0)
        wait_scatter()

    @pl.when(c == 0)
    def _():
        spmm(e1_hbm, w1_hbm)

    @pl.when(c == 1)
    def _():
        spmm(e2_hbm, w2_hbm)

    plsc.subcore_barrier()

    def emit(out_hbm):
        for j in range(4):
            m = s + NS * j

            @pl.when(m < N_GCHUNK)
            def _():
                b0 = m * EC
                pltpu.sync_copy(idx_hbm.at[pl.ds(b0, EC)], eb0.at[0])
                pltpu.async_copy(acc.at[eb0.at[0]], rows0, sem).wait()
                pltpu.sync_copy(rows0, out_hbm.at[pl.ds(b0, EC)])

        @pl.when(s == 14)
        def _():
            b0 = N_GCHUNK * EC
            pltpu.sync_copy(idx_hbm.at[pl.ds(b0, 40)], eb1.at[0, pl.ds(0, 40)])
            pltpu.async_copy(
                acc.at[eb1.at[0, pl.ds(0, 40)]], rows1.at[pl.ds(0, 40)],
                sem).wait()
            pltpu.sync_copy(rows1.at[pl.ds(0, 40)], out_hbm.at[pl.ds(b0, 40)])

    @pl.when(c == 0)
    def _():
        emit(g1_hbm)

    @pl.when(c == 1)
    def _():
        emit(g2_hbm)


@jax.jit
def _sc_spmm(x, e1, w1, e2, w2, idx):
    mesh = plsc.VectorSubcoreMesh(core_axis_name="c", subcore_axis_name="s")
    f = pl.kernel(
        _sc_body,
        out_type=(
            jax.ShapeDtypeStruct((B_IDX, D), jnp.float32),
            jax.ShapeDtypeStruct((B_IDX, D), jnp.float32),
        ),
        mesh=mesh,
        scratch_types=[
            pltpu.VMEM_SHARED((N_NODES, D), jnp.float32),
            pltpu.VMEM((ZR, D), jnp.float32),
            pltpu.VMEM((2, EC), jnp.int32),
            pltpu.VMEM((2, EC), jnp.int32),
            pltpu.VMEM((2, EC), jnp.int32),
            pltpu.VMEM((2, EC), jnp.int32),
            pltpu.VMEM((EC,), jnp.float32),
            pltpu.VMEM((EC,), jnp.float32),
            pltpu.VMEM((EC,), jnp.float32),
            pltpu.VMEM((EC,), jnp.float32),
            pltpu.VMEM((EC, D), jnp.float32),
            pltpu.VMEM((EC, D), jnp.float32),
            pltpu.VMEM((EC, D), jnp.float32),
            pltpu.SemaphoreType.DMA,
            pltpu.SemaphoreType.DMA,
            pltpu.SemaphoreType.DMA,
            pltpu.SemaphoreType.DMA,
            pltpu.SemaphoreType.DMA,
            pltpu.SemaphoreType.DMA,
            pltpu.SemaphoreType.DMA,
            pltpu.SemaphoreType.DMA,
            pltpu.SemaphoreType.DMA,
        ],
    )
    return f(x, e1, w1, e2, w2, idx)


def _mm_body(g1_ref, g2_ref, w1_ref, w2_ref, o_ref):
    o_ref[...] = (
        jnp.dot(g1_ref[...], w1_ref[...], preferred_element_type=jnp.float32)
        + jnp.dot(g2_ref[...], w2_ref[...], preferred_element_type=jnp.float32)
    )


@jax.jit
def _final_mm(g1, g2, W1, W2):
    return pl.pallas_call(
        _mm_body,
        grid=(5,),
        in_specs=[
            pl.BlockSpec((B_IDX // 5, D), lambda i: (i, 0)),
            pl.BlockSpec((B_IDX // 5, D), lambda i: (i, 0)),
            pl.BlockSpec((D, D), lambda i: (0, 0)),
            pl.BlockSpec((D, D), lambda i: (0, 0)),
        ],
        out_specs=pl.BlockSpec((B_IDX // 5, D), lambda i: (i, 0)),
        out_shape=jax.ShapeDtypeStruct((B_IDX, D), jnp.float32),
    )(g1, g2, W1, W2)


def kernel(idx, x, one_edge_index, one_edge_weight, two_edge_index,
           two_edge_weight, W1, W2):
    pad_idx = (
        jnp.arange(NS * 2 * EC, dtype=jnp.int32).reshape(NS, 2, EC)
        % N_NODES)

    def pack(edge_index):
        src = edge_index[0].reshape(NS, 250, EC)
        dst = edge_index[1].reshape(NS, 250, EC)
        e = jnp.stack([src, dst], axis=2)
        pad = jnp.stack([pad_idx, pad_idx], axis=2)
        e = jnp.concatenate([e, pad], axis=1)
        return e.reshape(NS * N_ECHUNK, 2, EC)

    def padw(w):
        w2 = w.reshape(NS, 250 * EC)
        pad = jnp.zeros((NS, 2 * EC), jnp.float32)
        return jnp.concatenate([w2, pad], axis=1).reshape(-1)

    e1 = pack(one_edge_index)
    e2 = pack(two_edge_index)
    w1 = padw(one_edge_weight)
    w2 = padw(two_edge_weight)
    g1, g2 = _sc_spmm(x, e1, w1, e2, w2, idx)
    return _final_mm(g1, g2, W1, W2)


# speedup vs baseline: 1.0444x; 1.0444x over previous
import jax
import jax.numpy as jnp
from jax import lax
from jax.experimental import pallas as pl
from jax.experimental.pallas import tpu as pltpu
from jax.experimental.pallas import tpu_sc as plsc

N_NODES = 10000
N_EDGES = 320000
D = 128
B_IDX = 5000

NS = 16
EC = 80
N_ECHUNK = 252
E_PER_TILE = N_ECHUNK * EC
ZR = 40
N_ZCHUNK = N_NODES // ZR
N_GCHUNK = 62


def _sc_body(x_hbm, e1_hbm, w1_hbm, e2_hbm, w2_hbm, idx_hbm,
             g1_hbm, g2_hbm,
             acc, zbuf, eb0, eb1, eb2, wb0, wb1, wb2, rows0, rows1, rows2,
             sem, semI0, semI1, semI2, semG0, semG1, semG2, semS):
    c = lax.axis_index("c")
    s = lax.axis_index("s")
    ebs = (eb0, eb1, eb2)
    wbs = (wb0, wb1, wb2)
    rowss = (rows0, rows1, rows2)
    semIs = (semI0, semI1, semI2)
    semGs = (semG0, semG1, semG2)

    def make_spmm(e_hbm, w_hbm):
        base = s * N_ECHUNK
        wbase = s * E_PER_TILE

        def start_idx(j, u):
            pltpu.async_copy(e_hbm.at[base + j], ebs[u], semIs[u])
            pltpu.async_copy(
                w_hbm.at[pl.ds(wbase + j * EC, EC)], wbs[u], semIs[u])

        def wait_idx(u):
            pltpu.make_async_copy(e_hbm.at[base], ebs[u], semIs[u]).wait()
            pltpu.make_async_copy(
                w_hbm.at[pl.ds(wbase, EC)], wbs[u], semIs[u]).wait()

        def start_gather(u):
            pltpu.async_copy(x_hbm.at[ebs[u].at[0]], rowss[u], semGs[u])

        def wait_gather(u):
            pltpu.make_async_copy(
                x_hbm.at[pl.ds(0, EC)], rowss[u], semGs[u]).wait()

        def wait_scatter():
            pltpu.make_async_copy(
                x_hbm.at[pl.ds(0, EC)], rowss[0], semS).wait()

        def sub(j, u):
            un = (u + 1) % 3

            @pl.when(j + 1 < N_ECHUNK)
            def _():
                wait_idx(un)
                start_gather(un)

            wait_gather(u)

            def scale16(g, carry2):
                wv = wbs[u][pl.ds(g * 16, 16)]
                for e16 in range(16):
                    e = g * 16 + e16
                    w = wv[e16]
                    for k in range(8):
                        sl = pl.ds(k * 16, 16)
                        rowss[u][e, sl] = rowss[u][e, sl] * w
                return carry2

            lax.fori_loop(0, EC // 16, scale16, 0)

            @pl.when(j >= 1)
            def _():
                wait_scatter()

            pltpu.async_copy(rowss[u], acc.at[ebs[u].at[1]], semS, add=True)

            @pl.when(j + 2 < N_ECHUNK)
            def _():
                start_idx(j + 2, (u + 2) % 3)

        def prologue():
            start_idx(0, 0)
            start_idx(1, 1)

        def launch():
            wait_idx(0)
            start_gather(0)

        def main():
            def triple(p, carry):
                sub(3 * p, 0)
                sub(3 * p + 1, 1)
                sub(3 * p + 2, 2)
                return carry

            lax.fori_loop(0, N_ECHUNK // 3, triple, 0)
            wait_scatter()

        return prologue, launch, main

    p1, l1, m1 = make_spmm(e1_hbm, w1_hbm)
    p2, l2, m2 = make_spmm(e2_hbm, w2_hbm)

    pl.when(c == 0)(p1)
    pl.when(c == 1)(p2)

    zero = jnp.zeros((16,), jnp.float32)

    def zrow(j, carry):
        for k in range(8):
            zbuf[j, pl.ds(k * 16, 16)] = zero
        return carry

    lax.fori_loop(0, ZR, zrow, 0)
    nz = (N_ZCHUNK + NS - 1) // NS
    for j in range(nz):
        m = s + NS * j

        @pl.when(m < N_ZCHUNK)
        def _():
            pltpu.async_copy(zbuf, acc.at[pl.ds(m * ZR, ZR)], semS)

    pl.when(c == 0)(l1)
    pl.when(c == 1)(l2)

    for j in range(nz):
        m = s + NS * j

        @pl.when(m < N_ZCHUNK)
        def _():
            pltpu.make_async_copy(x_hbm.at[pl.ds(0, ZR)], zbuf, semS).wait()

    plsc.subcore_barrier()

    pl.when(c == 0)(m1)
    pl.when(c == 1)(m2)

    plsc.subcore_barrier()

    def emit(out_hbm):
        i0, i1, i2, i3 = eb0.at[0], eb0.at[1], eb1.at[0], eb1.at[1]
        it = eb2.at[0, pl.ds(0, 40)]
        for j, ib in enumerate((i0, i1, i2)):
            pltpu.async_copy(idx_hbm.at[pl.ds((s + NS * j) * EC, EC)], ib, sem)

        @pl.when(s < 14)
        def _():
            pltpu.async_copy(
                idx_hbm.at[pl.ds((s + NS * 3) * EC, EC)], i3, sem)

        @pl.when(s == 14)
        def _():
            pltpu.async_copy(idx_hbm.at[pl.ds(N_GCHUNK * EC, 40)], it, sem)

        for ib in (i0, i1, i2):
            pltpu.make_async_copy(idx_hbm.at[pl.ds(0, EC)], ib, sem).wait()

        @pl.when(s < 14)
        def _():
            pltpu.make_async_copy(idx_hbm.at[pl.ds(0, EC)], i3, sem).wait()

        @pl.when(s == 14)
        def _():
            pltpu.make_async_copy(idx_hbm.at[pl.ds(0, 40)], it, sem).wait()

        pltpu.async_copy(acc.at[i0], rows0, semG0)
        pltpu.async_copy(acc.at[i1], rows1, semG1)
        pltpu.make_async_copy(x_hbm.at[pl.ds(0, EC)], rows0, semG0).wait()
        pltpu.sync_copy(rows0, out_hbm.at[pl.ds(s * EC, EC)])
        pltpu.async_copy(acc.at[i2], rows0, semG0)
        pltpu.make_async_copy(x_hbm.at[pl.ds(0, EC)], rows1, semG1).wait()
        pltpu.sync_copy(rows1, out_hbm.at[pl.ds((s + NS) * EC, EC)])

        @pl.when(s < 14)
        def _():
            pltpu.async_copy(acc.at[i3], rows1, semG1)

        @pl.when(s == 14)
        def _():
            pltpu.async_copy(acc.at[it], rows1.at[pl.ds(0, 40)], semG1)

        pltpu.make_async_copy(x_hbm.at[pl.ds(0, EC)], rows0, semG0).wait()
        pltpu.sync_copy(rows0, out_hbm.at[pl.ds((s + 2 * NS) * EC, EC)])

        @pl.when(s < 14)
        def _():
            pltpu.make_async_copy(x_hbm.at[pl.ds(0, EC)], rows1, semG1).wait()
            pltpu.sync_copy(rows1, out_hbm.at[pl.ds((s + 3 * NS) * EC, EC)])

        @pl.when(s == 14)
        def _():
            pltpu.make_async_copy(
                x_hbm.at[pl.ds(0, 40)], rows1.at[pl.ds(0, 40)], semG1).wait()
            pltpu.sync_copy(
                rows1.at[pl.ds(0, 40)], out_hbm.at[pl.ds(N_GCHUNK * EC, 40)])

    @pl.when(c == 0)
    def _():
        emit(g1_hbm)

    @pl.when(c == 1)
    def _():
        emit(g2_hbm)


@jax.jit
def _sc_spmm(x, e1, w1, e2, w2, idx):
    mesh = plsc.VectorSubcoreMesh(core_axis_name="c", subcore_axis_name="s")
    f = pl.kernel(
        _sc_body,
        out_type=(
            jax.ShapeDtypeStruct((B_IDX, D), jnp.float32),
            jax.ShapeDtypeStruct((B_IDX, D), jnp.float32),
        ),
        mesh=mesh,
        scratch_types=[
            pltpu.VMEM_SHARED((N_NODES, D), jnp.float32),
            pltpu.VMEM((ZR, D), jnp.float32),
            pltpu.VMEM((2, EC), jnp.int32),
            pltpu.VMEM((2, EC), jnp.int32),
            pltpu.VMEM((2, EC), jnp.int32),
            pltpu.VMEM((EC,), jnp.float32),
            pltpu.VMEM((EC,), jnp.float32),
            pltpu.VMEM((EC,), jnp.float32),
            pltpu.VMEM((EC, D), jnp.float32),
            pltpu.VMEM((EC, D), jnp.float32),
            pltpu.VMEM((EC, D), jnp.float32),
            pltpu.SemaphoreType.DMA,
            pltpu.SemaphoreType.DMA,
            pltpu.SemaphoreType.DMA,
            pltpu.SemaphoreType.DMA,
            pltpu.SemaphoreType.DMA,
            pltpu.SemaphoreType.DMA,
            pltpu.SemaphoreType.DMA,
            pltpu.SemaphoreType.DMA,
        ],
    )
    return f(x, e1, w1, e2, w2, idx)


def _mm_body(g1_ref, g2_ref, w1_ref, w2_ref, o_ref):
    o_ref[...] = (
        jnp.dot(g1_ref[...], w1_ref[...], preferred_element_type=jnp.float32)
        + jnp.dot(g2_ref[...], w2_ref[...], preferred_element_type=jnp.float32)
    )


@jax.jit
def _final_mm(g1, g2, W1, W2):
    return pl.pallas_call(
        _mm_body,
        grid=(5,),
        in_specs=[
            pl.BlockSpec((B_IDX // 5, D), lambda i: (i, 0)),
            pl.BlockSpec((B_IDX // 5, D), lambda i: (i, 0)),
            pl.BlockSpec((D, D), lambda i: (0, 0)),
            pl.BlockSpec((D, D), lambda i: (0, 0)),
        ],
        out_specs=pl.BlockSpec((B_IDX // 5, D), lambda i: (i, 0)),
        out_shape=jax.ShapeDtypeStruct((B_IDX, D), jnp.float32),
    )(g1, g2, W1, W2)


def kernel(idx, x, one_edge_index, one_edge_weight, two_edge_index,
           two_edge_weight, W1, W2):
    pad_idx = (
        jnp.arange(NS * 2 * EC, dtype=jnp.int32).reshape(NS, 2, EC)
        % N_NODES)

    def pack(edge_index):
        src = edge_index[0].reshape(NS, 250, EC)
        dst = edge_index[1].reshape(NS, 250, EC)
        e = jnp.stack([src, dst], axis=2)
        pad = jnp.stack([pad_idx, pad_idx], axis=2)
        e = jnp.concatenate([e, pad], axis=1)
        return e.reshape(NS * N_ECHUNK, 2, EC)

    def padw(w):
        w2 = w.reshape(NS, 250 * EC)
        pad = jnp.zeros((NS, 2 * EC), jnp.float32)
        return jnp.concatenate([w2, pad], axis=1).reshape(-1)

    e1 = pack(one_edge_index)
    e2 = pack(two_edge_index)
    w1 = padw(one_edge_weight)
    w2 = padw(two_edge_weight)
    g1, g2 = _sc_spmm(x, e1, w1, e2, w2, idx)
    return _final_mm(g1, g2, W1, W2)


# speedup vs baseline: 1.1192x; 1.0716x over previous
import jax
import jax.numpy as jnp
from jax import lax
from jax.experimental import pallas as pl
from jax.experimental.pallas import tpu as pltpu
from jax.experimental.pallas import tpu_sc as plsc

N_NODES = 10000
N_EDGES = 320000
D = 128
B_IDX = 5000

NS = 16
EC = 112
N_ECHUNK = 180
E_PER_TILE = N_ECHUNK * EC
ZR = 40
N_ZCHUNK = N_NODES // ZR
N_GCHUNK = 62


def _sc_body(x_hbm, e1_hbm, w1_hbm, e2_hbm, w2_hbm, idx_hbm,
             g1_hbm, g2_hbm,
             acc, zbuf, eb0, eb1, eb2, wb0, wb1, wb2, rows0, rows1, rows2,
             sem, semI0, semI1, semI2, semG0, semG1, semG2, semS):
    c = lax.axis_index("c")
    s = lax.axis_index("s")
    ebs = (eb0, eb1, eb2)
    wbs = (wb0, wb1, wb2)
    rowss = (rows0, rows1, rows2)
    semIs = (semI0, semI1, semI2)
    semGs = (semG0, semG1, semG2)

    def make_spmm(e_hbm, w_hbm):
        base = s * N_ECHUNK
        wbase = s * E_PER_TILE

        def start_idx(j, u):
            pltpu.async_copy(e_hbm.at[base + j], ebs[u], semIs[u])
            pltpu.async_copy(
                w_hbm.at[pl.ds(wbase + j * EC, EC)], wbs[u], semIs[u])

        def wait_idx(u):
            pltpu.make_async_copy(e_hbm.at[base], ebs[u], semIs[u]).wait()
            pltpu.make_async_copy(
                w_hbm.at[pl.ds(wbase, EC)], wbs[u], semIs[u]).wait()

        def start_gather(u):
            pltpu.async_copy(x_hbm.at[ebs[u].at[0]], rowss[u], semGs[u])

        def wait_gather(u):
            pltpu.make_async_copy(
                x_hbm.at[pl.ds(0, EC)], rowss[u], semGs[u]).wait()

        def wait_scatter():
            pltpu.make_async_copy(
                x_hbm.at[pl.ds(0, EC)], rowss[0], semS).wait()

        def sub(j, u):
            un = (u + 1) % 3

            @pl.when(j + 1 < N_ECHUNK)
            def _():
                wait_idx(un)
                start_gather(un)

            wait_gather(u)

            def scale16(g, carry2):
                wv = wbs[u][pl.ds(g * 16, 16)]
                for e16 in range(16):
                    e = g * 16 + e16
                    w = wv[e16]
                    for k in range(8):
                        sl = pl.ds(k * 16, 16)
                        rowss[u][e, sl] = rowss[u][e, sl] * w
                return carry2

            lax.fori_loop(0, EC // 16, scale16, 0)

            @pl.when(j >= 1)
            def _():
                wait_scatter()

            pltpu.async_copy(rowss[u], acc.at[ebs[u].at[1]], semS, add=True)

            @pl.when(j + 2 < N_ECHUNK)
            def _():
                start_idx(j + 2, (u + 2) % 3)

        def prologue():
            start_idx(0, 0)
            start_idx(1, 1)

        def launch():
            wait_idx(0)
            start_gather(0)

        def main():
            def triple(p, carry):
                sub(3 * p, 0)
                sub(3 * p + 1, 1)
                sub(3 * p + 2, 2)
                return carry

            lax.fori_loop(0, N_ECHUNK // 3, triple, 0)
            wait_scatter()

        return prologue, launch, main

    p1, l1, m1 = make_spmm(e1_hbm, w1_hbm)
    p2, l2, m2 = make_spmm(e2_hbm, w2_hbm)

    pl.when(c == 0)(p1)
    pl.when(c == 1)(p2)

    zero = jnp.zeros((16,), jnp.float32)

    def zrow(j, carry):
        for k in range(8):
            zbuf[j, pl.ds(k * 16, 16)] = zero
        return carry

    lax.fori_loop(0, ZR, zrow, 0)
    nz = (N_ZCHUNK + NS - 1) // NS
    for j in range(nz):
        m = s + NS * j

        @pl.when(m < N_ZCHUNK)
        def _():
            pltpu.async_copy(zbuf, acc.at[pl.ds(m * ZR, ZR)], semS)

    pl.when(c == 0)(l1)
    pl.when(c == 1)(l2)

    for j in range(nz):
        m = s + NS * j

        @pl.when(m < N_ZCHUNK)
        def _():
            pltpu.make_async_copy(x_hbm.at[pl.ds(0, ZR)], zbuf, semS).wait()

    plsc.subcore_barrier()

    pl.when(c == 0)(m1)
    pl.when(c == 1)(m2)

    plsc.subcore_barrier()

    def emit(out_hbm):
        GE = 80
        r0 = rows0.at[pl.ds(0, GE)]
        r1 = rows1.at[pl.ds(0, GE)]
        rt = rows1.at[pl.ds(0, 40)]
        i0, i1, i2, i3 = eb0.at[0, pl.ds(0, GE)], eb0.at[1, pl.ds(0, GE)], \
            eb1.at[0, pl.ds(0, GE)], eb1.at[1, pl.ds(0, GE)]
        it = eb2.at[0, pl.ds(0, 40)]
        for j, ib in enumerate((i0, i1, i2)):
            pltpu.async_copy(idx_hbm.at[pl.ds((s + NS * j) * GE, GE)], ib, sem)

        @pl.when(s < 14)
        def _():
            pltpu.async_copy(
                idx_hbm.at[pl.ds((s + NS * 3) * GE, GE)], i3, sem)

        @pl.when(s == 14)
        def _():
            pltpu.async_copy(idx_hbm.at[pl.ds(N_GCHUNK * GE, 40)], it, sem)

        for ib in (i0, i1, i2):
            pltpu.make_async_copy(idx_hbm.at[pl.ds(0, GE)], ib, sem).wait()

        @pl.when(s < 14)
        def _():
            pltpu.make_async_copy(idx_hbm.at[pl.ds(0, GE)], i3, sem).wait()

        @pl.when(s == 14)
        def _():
            pltpu.make_async_copy(idx_hbm.at[pl.ds(0, 40)], it, sem).wait()

        pltpu.async_copy(acc.at[i0], r0, semG0)
        pltpu.async_copy(acc.at[i1], r1, semG1)
        pltpu.make_async_copy(x_hbm.at[pl.ds(0, GE)], r0, semG0).wait()
        pltpu.sync_copy(r0, out_hbm.at[pl.ds(s * GE, GE)])
        pltpu.async_copy(acc.at[i2], r0, semG0)
        pltpu.make_async_copy(x_hbm.at[pl.ds(0, GE)], r1, semG1).wait()
        pltpu.sync_copy(r1, out_hbm.at[pl.ds((s + NS) * GE, GE)])

        @pl.when(s < 14)
        def _():
            pltpu.async_copy(acc.at[i3], r1, semG1)

        @pl.when(s == 14)
        def _():
            pltpu.async_copy(acc.at[it], rt, semG1)

        pltpu.make_async_copy(x_hbm.at[pl.ds(0, GE)], r0, semG0).wait()
        pltpu.sync_copy(r0, out_hbm.at[pl.ds((s + 2 * NS) * GE, GE)])

        @pl.when(s < 14)
        def _():
            pltpu.make_async_copy(x_hbm.at[pl.ds(0, GE)], r1, semG1).wait()
            pltpu.sync_copy(r1, out_hbm.at[pl.ds((s + 3 * NS) * GE, GE)])

        @pl.when(s == 14)
        def _():
            pltpu.make_async_copy(x_hbm.at[pl.ds(0, 40)], rt, semG1).wait()
            pltpu.sync_copy(rt, out_hbm.at[pl.ds(N_GCHUNK * GE, 40)])

    @pl.when(c == 0)
    def _():
        emit(g1_hbm)

    @pl.when(c == 1)
    def _():
        emit(g2_hbm)


@jax.jit
def _sc_spmm(x, e1, w1, e2, w2, idx):
    mesh = plsc.VectorSubcoreMesh(core_axis_name="c", subcore_axis_name="s")
    f = pl.kernel(
        _sc_body,
        out_type=(
            jax.ShapeDtypeStruct((B_IDX, D), jnp.float32),
            jax.ShapeDtypeStruct((B_IDX, D), jnp.float32),
        ),
        mesh=mesh,
        scratch_types=[
            pltpu.VMEM_SHARED((N_NODES, D), jnp.float32),
            pltpu.VMEM((ZR, D), jnp.float32),
            pltpu.VMEM((2, EC), jnp.int32),
            pltpu.VMEM((2, EC), jnp.int32),
            pltpu.VMEM((2, EC), jnp.int32),
            pltpu.VMEM((EC,), jnp.float32),
            pltpu.VMEM((EC,), jnp.float32),
            pltpu.VMEM((EC,), jnp.float32),
            pltpu.VMEM((EC, D), jnp.float32),
            pltpu.VMEM((EC, D), jnp.float32),
            pltpu.VMEM((EC, D), jnp.float32),
            pltpu.SemaphoreType.DMA,
            pltpu.SemaphoreType.DMA,
            pltpu.SemaphoreType.DMA,
            pltpu.SemaphoreType.DMA,
            pltpu.SemaphoreType.DMA,
            pltpu.SemaphoreType.DMA,
            pltpu.SemaphoreType.DMA,
            pltpu.SemaphoreType.DMA,
        ],
    )
    return f(x, e1, w1, e2, w2, idx)


def _mm_body(g1_ref, g2_ref, w1_ref, w2_ref, o_ref):
    o_ref[...] = (
        jnp.dot(g1_ref[...], w1_ref[...], preferred_element_type=jnp.float32)
        + jnp.dot(g2_ref[...], w2_ref[...], preferred_element_type=jnp.float32)
    )


@jax.jit
def _final_mm(g1, g2, W1, W2):
    return pl.pallas_call(
        _mm_body,
        grid=(5,),
        in_specs=[
            pl.BlockSpec((B_IDX // 5, D), lambda i: (i, 0)),
            pl.BlockSpec((B_IDX // 5, D), lambda i: (i, 0)),
            pl.BlockSpec((D, D), lambda i: (0, 0)),
            pl.BlockSpec((D, D), lambda i: (0, 0)),
        ],
        out_specs=pl.BlockSpec((B_IDX // 5, D), lambda i: (i, 0)),
        out_shape=jax.ShapeDtypeStruct((B_IDX, D), jnp.float32),
    )(g1, g2, W1, W2)


def kernel(idx, x, one_edge_index, one_edge_weight, two_edge_index,
           two_edge_weight, W1, W2):
    n_real = N_EDGES // NS
    n_pad = E_PER_TILE - n_real
    pad_idx = (
        jnp.arange(NS * n_pad, dtype=jnp.int32).reshape(NS, n_pad)
        % N_NODES)

    def pack(edge_index):
        src = jnp.concatenate(
            [edge_index[0].reshape(NS, n_real), pad_idx], axis=1)
        dst = jnp.concatenate(
            [edge_index[1].reshape(NS, n_real), pad_idx], axis=1)
        src = src.reshape(NS, N_ECHUNK, EC)
        dst = dst.reshape(NS, N_ECHUNK, EC)
        e = jnp.stack([src, dst], axis=2)
        return e.reshape(NS * N_ECHUNK, 2, EC)

    def padw(w):
        w2 = w.reshape(NS, n_real)
        pad = jnp.zeros((NS, n_pad), jnp.float32)
        return jnp.concatenate([w2, pad], axis=1).reshape(-1)

    e1 = pack(one_edge_index)
    e2 = pack(two_edge_index)
    w1 = padw(one_edge_weight)
    w2 = padw(two_edge_weight)
    g1, g2 = _sc_spmm(x, e1, w1, e2, w2, idx)
    return _final_mm(g1, g2, W1, W2)


# speedup vs baseline: 1.1280x; 1.0079x over previous
import jax
import jax.numpy as jnp
from jax import lax
from jax.experimental import pallas as pl
from jax.experimental.pallas import tpu as pltpu
from jax.experimental.pallas import tpu_sc as plsc

N_NODES = 10000
N_EDGES = 320000
D = 128
B_IDX = 5000

NS = 16
EC = 128
N_ECHUNK = 159
E_PER_TILE = N_ECHUNK * EC
ZR = 40
N_ZCHUNK = N_NODES // ZR
N_GCHUNK = 62


def _sc_body(x_hbm, e1_hbm, w1_hbm, e2_hbm, w2_hbm, idx_hbm,
             g1_hbm, g2_hbm,
             acc, eb0, eb1, eb2, wb0, wb1, wb2, rows0, rows1, rows2,
             sem, semI0, semI1, semI2, semG0, semG1, semG2, semS):
    c = lax.axis_index("c")
    s = lax.axis_index("s")
    ebs = (eb0, eb1, eb2)
    wbs = (wb0, wb1, wb2)
    rowss = (rows0, rows1, rows2)
    semIs = (semI0, semI1, semI2)
    semGs = (semG0, semG1, semG2)

    def make_spmm(e_hbm, w_hbm):
        base = s * N_ECHUNK
        wbase = s * E_PER_TILE

        def start_idx(j, u):
            pltpu.async_copy(e_hbm.at[base + j], ebs[u], semIs[u])
            pltpu.async_copy(
                w_hbm.at[pl.ds(wbase + j * EC, EC)], wbs[u], semIs[u])

        def wait_idx(u):
            pltpu.make_async_copy(e_hbm.at[base], ebs[u], semIs[u]).wait()
            pltpu.make_async_copy(
                w_hbm.at[pl.ds(wbase, EC)], wbs[u], semIs[u]).wait()

        def start_gather(u):
            pltpu.async_copy(x_hbm.at[ebs[u].at[0]], rowss[u], semGs[u])

        def wait_gather(u):
            pltpu.make_async_copy(
                x_hbm.at[pl.ds(0, EC)], rowss[u], semGs[u]).wait()

        def wait_scatter():
            pltpu.make_async_copy(
                x_hbm.at[pl.ds(0, EC)], rowss[0], semS).wait()

        def sub(j, u):
            un = (u + 1) % 3

            @pl.when(j + 1 < N_ECHUNK)
            def _():
                wait_idx(un)
                start_gather(un)

            wait_gather(u)

            def scale16(g, carry2):
                wv = wbs[u][pl.ds(g * 16, 16)]
                for e16 in range(16):
                    e = g * 16 + e16
                    w = wv[e16]
                    for k in range(8):
                        sl = pl.ds(k * 16, 16)
                        rowss[u][e, sl] = rowss[u][e, sl] * w
                return carry2

            lax.fori_loop(0, EC // 16, scale16, 0)

            @pl.when(j >= 1)
            def _():
                wait_scatter()

            pltpu.async_copy(rowss[u], acc.at[ebs[u].at[1]], semS, add=True)

            @pl.when(j + 2 < N_ECHUNK)
            def _():
                start_idx(j + 2, (u + 2) % 3)

        def prologue():
            start_idx(0, 0)
            start_idx(1, 1)

        def launch():
            wait_idx(0)
            start_gather(0)

        def main():
            def triple(p, carry):
                sub(3 * p, 0)
                sub(3 * p + 1, 1)
                sub(3 * p + 2, 2)
                return carry

            lax.fori_loop(0, N_ECHUNK // 3, triple, 0)
            wait_scatter()

        return prologue, launch, main

    p1, l1, m1 = make_spmm(e1_hbm, w1_hbm)
    p2, l2, m2 = make_spmm(e2_hbm, w2_hbm)

    pl.when(c == 0)(p1)
    pl.when(c == 1)(p2)

    zero = jnp.zeros((16,), jnp.float32)

    def zrow(j, carry):
        for k in range(8):
            rows2[j, pl.ds(k * 16, 16)] = zero
        return carry

    lax.fori_loop(0, ZR, zrow, 0)
    nz = (N_ZCHUNK + NS - 1) // NS
    for j in range(nz):
        m = s + NS * j

        @pl.when(m < N_ZCHUNK)
        def _():
            pltpu.async_copy(
                rows2.at[pl.ds(0, ZR)], acc.at[pl.ds(m * ZR, ZR)], semS)

    pl.when(c == 0)(l1)
    pl.when(c == 1)(l2)

    for j in range(nz):
        m = s + NS * j

        @pl.when(m < N_ZCHUNK)
        def _():
            pltpu.make_async_copy(
                x_hbm.at[pl.ds(0, ZR)], rows2.at[pl.ds(0, ZR)], semS).wait()

    plsc.subcore_barrier()

    pl.when(c == 0)(m1)
    pl.when(c == 1)(m2)

    plsc.subcore_barrier()

    def emit(out_hbm):
        GE = 80
        r0 = rows0.at[pl.ds(0, GE)]
        r1 = rows1.at[pl.ds(0, GE)]
        rt = rows1.at[pl.ds(0, 40)]
        i0, i1, i2, i3 = eb0.at[0, pl.ds(0, GE)], eb0.at[1, pl.ds(0, GE)], \
            eb1.at[0, pl.ds(0, GE)], eb1.at[1, pl.ds(0, GE)]
        it = eb2.at[0, pl.ds(0, 40)]
        for j, ib in enumerate((i0, i1, i2)):
            pltpu.async_copy(idx_hbm.at[pl.ds((s + NS * j) * GE, GE)], ib, sem)

        @pl.when(s < 14)
        def _():
            pltpu.async_copy(
                idx_hbm.at[pl.ds((s + NS * 3) * GE, GE)], i3, sem)

        @pl.when(s == 14)
        def _():
            pltpu.async_copy(idx_hbm.at[pl.ds(N_GCHUNK * GE, 40)], it, sem)

        for ib in (i0, i1, i2):
            pltpu.make_async_copy(idx_hbm.at[pl.ds(0, GE)], ib, sem).wait()

        @pl.when(s < 14)
        def _():
            pltpu.make_async_copy(idx_hbm.at[pl.ds(0, GE)], i3, sem).wait()

        @pl.when(s == 14)
        def _():
            pltpu.make_async_copy(idx_hbm.at[pl.ds(0, 40)], it, sem).wait()

        pltpu.async_copy(acc.at[i0], r0, semG0)
        pltpu.async_copy(acc.at[i1], r1, semG1)
        pltpu.make_async_copy(x_hbm.at[pl.ds(0, GE)], r0, semG0).wait()
        pltpu.sync_copy(r0, out_hbm.at[pl.ds(s * GE, GE)])
        pltpu.async_copy(acc.at[i2], r0, semG0)
        pltpu.make_async_copy(x_hbm.at[pl.ds(0, GE)], r1, semG1).wait()
        pltpu.sync_copy(r1, out_hbm.at[pl.ds((s + NS) * GE, GE)])

        @pl.when(s < 14)
        def _():
            pltpu.async_copy(acc.at[i3], r1, semG1)

        @pl.when(s == 14)
        def _():
            pltpu.async_copy(acc.at[it], rt, semG1)

        pltpu.make_async_copy(x_hbm.at[pl.ds(0, GE)], r0, semG0).wait()
        pltpu.sync_copy(r0, out_hbm.at[pl.ds((s + 2 * NS) * GE, GE)])

        @pl.when(s < 14)
        def _():
            pltpu.make_async_copy(x_hbm.at[pl.ds(0, GE)], r1, semG1).wait()
            pltpu.sync_copy(r1, out_hbm.at[pl.ds((s + 3 * NS) * GE, GE)])

        @pl.when(s == 14)
        def _():
            pltpu.make_async_copy(x_hbm.at[pl.ds(0, 40)], rt, semG1).wait()
            pltpu.sync_copy(rt, out_hbm.at[pl.ds(N_GCHUNK * GE, 40)])

    @pl.when(c == 0)
    def _():
        emit(g1_hbm)

    @pl.when(c == 1)
    def _():
        emit(g2_hbm)


@jax.jit
def _sc_spmm(x, e1, w1, e2, w2, idx):
    mesh = plsc.VectorSubcoreMesh(core_axis_name="c", subcore_axis_name="s")
    f = pl.kernel(
        _sc_body,
        out_type=(
            jax.ShapeDtypeStruct((B_IDX, D), jnp.float32),
            jax.ShapeDtypeStruct((B_IDX, D), jnp.float32),
        ),
        mesh=mesh,
        scratch_types=[
            pltpu.VMEM_SHARED((N_NODES, D), jnp.float32),
            pltpu.VMEM((2, EC), jnp.int32),
            pltpu.VMEM((2, EC), jnp.int32),
            pltpu.VMEM((2, EC), jnp.int32),
            pltpu.VMEM((EC,), jnp.float32),
            pltpu.VMEM((EC,), jnp.float32),
            pltpu.VMEM((EC,), jnp.float32),
            pltpu.VMEM((EC, D), jnp.float32),
            pltpu.VMEM((EC, D), jnp.float32),
            pltpu.VMEM((EC, D), jnp.float32),
            pltpu.SemaphoreType.DMA,
            pltpu.SemaphoreType.DMA,
            pltpu.SemaphoreType.DMA,
            pltpu.SemaphoreType.DMA,
            pltpu.SemaphoreType.DMA,
            pltpu.SemaphoreType.DMA,
            pltpu.SemaphoreType.DMA,
            pltpu.SemaphoreType.DMA,
        ],
    )
    return f(x, e1, w1, e2, w2, idx)


def _mm_body(g1_ref, g2_ref, w1_ref, w2_ref, o_ref):
    o_ref[...] = (
        jnp.dot(g1_ref[...], w1_ref[...], preferred_element_type=jnp.float32)
        + jnp.dot(g2_ref[...], w2_ref[...], preferred_element_type=jnp.float32)
    )


@jax.jit
def _final_mm(g1, g2, W1, W2):
    return pl.pallas_call(
        _mm_body,
        grid=(5,),
        in_specs=[
            pl.BlockSpec((B_IDX // 5, D), lambda i: (i, 0)),
            pl.BlockSpec((B_IDX // 5, D), lambda i: (i, 0)),
            pl.BlockSpec((D, D), lambda i: (0, 0)),
            pl.BlockSpec((D, D), lambda i: (0, 0)),
        ],
        out_specs=pl.BlockSpec((B_IDX // 5, D), lambda i: (i, 0)),
        out_shape=jax.ShapeDtypeStruct((B_IDX, D), jnp.float32),
    )(g1, g2, W1, W2)


def kernel(idx, x, one_edge_index, one_edge_weight, two_edge_index,
           two_edge_weight, W1, W2):
    n_real = N_EDGES // NS
    n_pad = E_PER_TILE - n_real
    pad_idx = (
        jnp.arange(NS * n_pad, dtype=jnp.int32).reshape(NS, n_pad)
        % N_NODES)

    def pack(edge_index):
        src = jnp.concatenate(
            [edge_index[0].reshape(NS, n_real), pad_idx], axis=1)
        dst = jnp.concatenate(
            [edge_index[1].reshape(NS, n_real), pad_idx], axis=1)
        src = src.reshape(NS, N_ECHUNK, EC)
        dst = dst.reshape(NS, N_ECHUNK, EC)
        e = jnp.stack([src, dst], axis=2)
        return e.reshape(NS * N_ECHUNK, 2, EC)

    def padw(w):
        w2 = w.reshape(NS, n_real)
        pad = jnp.zeros((NS, n_pad), jnp.float32)
        return jnp.concatenate([w2, pad], axis=1).reshape(-1)

    e1 = pack(one_edge_index)
    e2 = pack(two_edge_index)
    w1 = padw(one_edge_weight)
    w2 = padw(two_edge_weight)
    g1, g2 = _sc_spmm(x, e1, w1, e2, w2, idx)
    return _final_mm(g1, g2, W1, W2)
